# baseline (speedup 1.0000x reference)
import functools

import numpy as np
import jax
import jax.numpy as jnp
from jax import lax
from jax.experimental import pallas as pl
from jax.experimental.pallas import tpu as pltpu
from jax.experimental.pallas import tpu_sc as plsc

NC = 2
NS = 16
NW = NC * NS
CH = 128
L = 16


def _pack_perm(H):
    perm = np.empty(H, np.int64)
    for g in range(H // 32):
        for i in range(16):
            perm[32 * g + i] = 32 * g + 2 * i
            perm[32 * g + 16 + i] = 32 * g + 2 * i + 1
    inv = np.argsort(perm)
    return perm, inv


def _gather_kernel(E, N, H):
    nchunk = E // CH
    iters = -(-nchunk // NW)
    mesh = plsc.VectorSubcoreMesh(core_axis_name="c", subcore_axis_name="s")

    @functools.partial(
        pl.kernel,
        mesh=mesh,
        out_type=(
            jax.ShapeDtypeStruct((E, H), jnp.float32),
            jax.ShapeDtypeStruct((nchunk, 4, CH), jnp.float32),
        ),
        scratch_types=[
            pltpu.VMEM((N * 3,), jnp.float32),
            pltpu.VMEM((CH,), jnp.int32),
            pltpu.VMEM((CH,), jnp.int32),
            pltpu.VMEM((CH,), jnp.int32),
            pltpu.VMEM((CH,), jnp.int32),
            pltpu.VMEM((CH, H), jnp.float32),
            pltpu.VMEM((CH, H), jnp.float32),
            pltpu.VMEM((CH, H), jnp.float32),
            pltpu.VMEM((CH, H), jnp.float32),
            pltpu.VMEM((4, CH), jnp.float32),
            pltpu.SemaphoreType.DMA,
            pltpu.SemaphoreType.DMA,
            pltpu.SemaphoreType.DMA,
            pltpu.SemaphoreType.DMA,
        ],
        compiler_params=pltpu.CompilerParams(needs_layout_passes=False),
    )
    def k(p_hbm, q_hbm, x_hbm, src_hbm, dst_hbm,
          z1_out, geo_out,
          x_v, src_v0, dst_v0, src_v1, dst_v1,
          z1_v0, qd_v0, z1_v1, qd_v1,
          geo_v, s0, s1, s2, s3):
        wid = lax.axis_index("s") * NC + lax.axis_index("c")
        pltpu.sync_copy(x_hbm, x_v)
        bufs = ((src_v0, dst_v0, z1_v0, qd_v0, s0, s1),
                (src_v1, dst_v1, z1_v1, qd_v1, s2, s3))

        def issue(kk, sv, dv, zv, qv, sp, sq):
            base = kk * CH
            pltpu.sync_copy(src_hbm.at[pl.ds(base, CH)], sv)
            pltpu.sync_copy(dst_hbm.at[pl.ds(base, CH)], dv)
            pltpu.async_copy(p_hbm.at[sv], zv, sp)
            pltpu.async_copy(q_hbm.at[dv], qv, sq)

        @pl.when(wid < nchunk)
        def _():
            issue(wid, *bufs[0])

        def grp(g, carry):
            for b in (0, 1):
                j = g * 2 + b
                kk = wid + j * NW

                @pl.when(kk < nchunk)
                def _(b=b, kk=kk):
                    sv, dv, zv, qv, sp, sq = bufs[b]
                    kk2 = kk + NW

                    @pl.when(kk2 < nchunk)
                    def _():
                        issue(kk2, *bufs[1 - b])

                    base = kk * CH
                    for i in range(CH // L):
                        sl = pl.ds(i * L, L)
                        srcs = sv[sl] * 3
                        dsts = dv[sl] * 3
                        d = []
                        for c in range(3):
                            xs_c = plsc.load_gather(x_v, [srcs + c])
                            xd_c = plsc.load_gather(x_v, [dsts + c])
                            d.append(xs_c - xd_c)
                        geo_v[0, sl] = (d[0] * d[0] + d[1] * d[1]
                                        + d[2] * d[2]) / 100.0
                        geo_v[1, sl] = d[0]
                        geo_v[2, sl] = d[1]
                        geo_v[3, sl] = d[2]
                    pltpu.make_async_copy(p_hbm.at[sv], zv, sp).wait()
                    pltpu.make_async_copy(q_hbm.at[dv], qv, sq).wait()

                    def addpack(r, carry2):
                        for gg in range(H // L):
                            sl2 = pl.ds(L * gg, L)
                            plsc.addupdate(zv.at[r, sl2], qv[r, sl2])
                        return carry2

                    lax.fori_loop(0, CH, addpack, 0)
                    pltpu.sync_copy(zv, z1_out.at[pl.ds(base, CH)])
                    pltpu.sync_copy(geo_v, geo_out.at[kk])

            return carry

        lax.fori_loop(0, (iters + 1) // 2, grp, 0)

    return k


def _scatter_kernel(E, N, H):
    nchunk = E // CH
    iters = -(-nchunk // NW)
    rpt = (N // NS) // 8 * 8
    tail = N - rpt * NS
    NP3 = -(-(N * 3) // 1024) * 1024
    mesh = plsc.VectorSubcoreMesh(core_axis_name="c", subcore_axis_name="s")

    @functools.partial(
        pl.kernel,
        mesh=mesh,
        out_type=(
            jax.ShapeDtypeStruct((NC, N, H), jnp.float32),
            jax.ShapeDtypeStruct((NW, NP3), jnp.float32),
        ),
        scratch_types=[
            pltpu.VMEM((CH,), jnp.int32),
            pltpu.VMEM((CH,), jnp.int32),
            pltpu.VMEM((CH, H), jnp.float32),
            pltpu.VMEM((CH,), jnp.float32),
            pltpu.VMEM((CH,), jnp.float32),
            pltpu.VMEM((4, CH), jnp.float32),
            pltpu.VMEM((4, CH), jnp.float32),
            pltpu.VMEM((NP3,), jnp.float32),
            pltpu.VMEM_SHARED((N, H), jnp.float32),
            pltpu.SemaphoreType.DMA,
            pltpu.SemaphoreType.DMA,
            pltpu.SemaphoreType.DMA,
            pltpu.SemaphoreType.DMA,
            pltpu.SemaphoreType.DMA,
            pltpu.SemaphoreType.DMA,
            pltpu.SemaphoreType.DMA,
        ],
        compiler_params=pltpu.CompilerParams(needs_layout_passes=False),
    )
    def k(m_hbm, cw_hbm, geo_hbm, dst_hbm, z_h_hbm, z_x_hbm,
          msg_out, cda_out,
          dst_v0, dst_v1, m_v, cw_v0, cw_v1, geo_v0, geo_v1,
          acc_v, msg_sh, sd0, sc0, sg0, sd1, sc1, sg1, sm):
        cid = lax.axis_index("c")
        sid = lax.axis_index("s")
        wid = sid * NC + cid
        rows = pl.ds(sid * rpt, rpt)
        trows = pl.ds(rpt * NS, tail)
        pltpu.sync_copy(z_h_hbm.at[rows], msg_sh.at[rows])

        @pl.when(sid == 0)
        def _():
            pltpu.sync_copy(z_h_hbm.at[trows], msg_sh.at[trows])

        pltpu.sync_copy(z_x_hbm, acc_v)
        plsc.subcore_barrier()
        bufs = ((dst_v0, cw_v0, geo_v0, sd0, sc0, sg0),
                (dst_v1, cw_v1, geo_v1, sd1, sc1, sg1))

        def issue(kk, dv, cv, gv, sd, sc, sg):
            base = kk * CH
            pltpu.async_copy(dst_hbm.at[pl.ds(base, CH)], dv, sd)
            pltpu.async_copy(cw_hbm.at[kk], cv, sc)
            pltpu.async_copy(geo_hbm.at[kk], gv, sg)

        def issue_m(kk):
            pltpu.async_copy(m_hbm.at[pl.ds(kk * CH, CH)], m_v, sm)

        @pl.when(wid < nchunk)
        def _():
            issue(wid, *bufs[0])
            issue_m(wid)

        def grp(g, carry):
            for b in (0, 1):
                j = g * 2 + b
                kk = wid + j * NW

                @pl.when(kk < nchunk)
                def _(b=b, kk=kk):
                    dv, cv, gv, sd, sc, sg = bufs[b]
                    kk2 = kk + NW

                    @pl.when(kk2 < nchunk)
                    def _():
                        issue(kk2, *bufs[1 - b])

                    base = kk * CH
                    pltpu.make_async_copy(
                        dst_hbm.at[pl.ds(base, CH)], dv, sd).wait()
                    pltpu.make_async_copy(cw_hbm.at[kk], cv, sc).wait()
                    pltpu.make_async_copy(geo_hbm.at[kk], gv, sg).wait()
                    pltpu.make_async_copy(
                        m_hbm.at[pl.ds(base, CH)], m_v, sm).wait()
                    pltpu.sync_copy(m_v, msg_sh.at[dv], add=True)

                    @pl.when(kk2 < nchunk)
                    def _():
                        issue_m(kk2)

                    for i in range(CH // L):
                        sl = pl.ds(i * L, L)
                        dsts = dv[sl] * 3
                        cws = cv[sl]
                        plsc.addupdate_scatter(acc_v, [dsts],
                                               gv[1, sl] * cws)
                        plsc.addupdate_scatter(acc_v, [dsts + 1],
                                               gv[2, sl] * cws)
                        plsc.addupdate_scatter(acc_v, [dsts + 2],
                                               gv[3, sl] * cws)

            return carry

        lax.fori_loop(0, (iters + 1) // 2, grp, 0)
        pltpu.sync_copy(acc_v, cda_out.at[wid])
        plsc.subcore_barrier()
        pltpu.sync_copy(msg_sh.at[rows], msg_out.at[cid].at[rows])

        @pl.when(sid == 0)
        def _():
            pltpu.sync_copy(msg_sh.at[trows], msg_out.at[cid].at[trows])

    return k


def _prep_body(h, Wa, Wb, p_out, q_out):
    p_out[...] = jnp.dot(h[...].astype(jnp.bfloat16), Wa[...],
                         preferred_element_type=jnp.float32)
    q_out[...] = jnp.dot(h[...].astype(jnp.bfloat16), Wb[...],
                         preferred_element_type=jnp.float32)


def _edge_mlp_body(z1, ea, dist, wc, Wd, be1, We2,
                   be2, Wx1, bx1, Wx2, bx2, m_out, cw_out):
    bf = jnp.bfloat16
    cr, ch = dist.shape[0], dist.shape[2]
    et = z1.shape[0]
    rows25 = lax.broadcasted_iota(jnp.int32, (et, cr), 0)
    cols25 = lax.broadcasted_iota(jnp.int32, (et, cr), 1)
    P1 = (rows25 // ch == cols25).astype(jnp.float32)
    rows1 = lax.broadcasted_iota(jnp.int32, (et, ch), 0)
    cols1 = lax.broadcasted_iota(jnp.int32, (et, ch), 1)
    Q = (rows1 % ch == cols1).astype(jnp.float32)
    rowsT = lax.broadcasted_iota(jnp.int32, (cr, et), 0)
    colsT = lax.broadcasted_iota(jnp.int32, (cr, et), 1)
    P1T = (colsT // ch == rowsT).astype(jnp.float32)

    D = dist[...][:, 0, :]
    dist_col = jnp.sum(
        jnp.dot(P1, D, preferred_element_type=jnp.float32) * Q,
        axis=1, keepdims=True)
    z = z1[...].astype(jnp.float32)
    z = z + jnp.dot(ea[...].astype(bf), Wd[...],
                    preferred_element_type=jnp.float32)
    z = z + dist_col * wc[...] + be1[...]
    m = z * jax.nn.sigmoid(z)
    z2 = jnp.dot(m.astype(bf), We2[...],
                 preferred_element_type=jnp.float32) + be2[...]
    mij = z2 * jax.nn.sigmoid(z2)
    z3 = jnp.dot(mij.astype(bf), Wx1[...],
                 preferred_element_type=jnp.float32) + bx1[...]
    t = z3 * jax.nn.sigmoid(z3)
    z4 = jnp.dot(t.astype(bf), Wx2[...],
                 preferred_element_type=jnp.float32) + bx2[...]
    cw = jnp.tanh(z4)
    m_out[...] = mij
    cw_out[...] = jnp.dot(P1T, cw * Q,
                          preferred_element_type=jnp.float32).reshape(
                              cw_out.shape)


def _cred_body(cdp, out):
    out[...] = jnp.sum(cdp[...], axis=0).reshape(out.shape)


def _node_body(h, msgpa, msgpb, cda, x3, fm, Wh1a, Wh1b, bh1, Wh2, bh2,
               hn_out, xn_out):
    bf = jnp.bfloat16
    msg = msgpa[0] + msgpa[1] + msgpb[0] + msgpb[1]
    z = (jnp.dot(h[...].astype(bf), Wh1a[...],
                 preferred_element_type=jnp.float32)
         + jnp.dot(msg.astype(bf), Wh1b[...],
                   preferred_element_type=jnp.float32)
         + bh1[...])
    t = z * jax.nn.sigmoid(z)
    hn_out[...] = h[...] + jnp.dot(t.astype(bf), Wh2[...],
                                   preferred_element_type=jnp.float32) + bh2[...]
    umask = 1.0 - fm[...].astype(jnp.float32)
    xn_out[...] = x3[...] + cda[...] * umask


def kernel(h, x, edge_index, edge_attr, fixed_mask,
           We1, be1, We2, be2, Wx1, bx1, Wx2, bx2, Wh1, bh1, Wh2, bh2):
    N, H = h.shape
    E = edge_index.shape[1]
    ED = edge_attr.shape[1]
    src = edge_index[0]
    dst = edge_index[1]
    bf = jnp.bfloat16

    full = lambda shape: pl.BlockSpec(shape, lambda i: (0,) * len(shape))

    NT = 2000
    gn = N // NT
    nblk = lambda w: pl.BlockSpec((NT, w), lambda i: (i, 0))
    Wa = We1[:H].astype(bf)
    Wb = We1[H:2 * H].astype(bf)
    P, Qm = pl.pallas_call(
        _prep_body,
        grid=(gn,),
        in_specs=[nblk(H), full((H, H)), full((H, H))],
        out_specs=[nblk(H), nblk(H)],
        out_shape=[
            jax.ShapeDtypeStruct((N, H), jnp.float32),
            jax.ShapeDtypeStruct((N, H), jnp.float32),
        ],
    )(h, Wa, Wb)

    NSPLIT = 2
    E2 = E // NSPLIT
    nchunk2 = E2 // CH
    xflat = x.reshape(-1)
    zh = jnp.zeros((N, H), jnp.float32)
    NP3 = -(-(N * 3) // 1024) * 1024
    zx = jnp.zeros((NP3,), jnp.float32)

    wc = We1[2 * H:2 * H + 1]
    Wd = We1[2 * H + 1:].astype(bf)
    be1p = be1.reshape(1, H)
    ET = 3200
    ge = E2 // ET
    CR = ET // CH
    eblk = lambda w: pl.BlockSpec((ET, w), lambda i: (i, 0))
    sblk = pl.BlockSpec((1, CR, CH), lambda i: (i, 0, 0))

    gatherk = _gather_kernel(E2, N, H)
    scatterk = _scatter_kernel(E2, N, H)

    msgps, cdps = [], []
    for half in range(NSPLIT):
        sl = slice(half * E2, (half + 1) * E2)
        srch, dsth = src[sl], dst[sl]

        z1, geo = gatherk(P, Qm, xflat, srch, dsth)

        gblk = pl.BlockSpec((CR, 4, CH), lambda i: (i, 0, 0))
        mij, cwm = pl.pallas_call(
            _edge_mlp_body,
            grid=(ge,),
            in_specs=[
                eblk(H), eblk(ED), gblk,
                full((1, H)), full((ED, H)),
                full((1, H)), full((H, H)), full((1, H)),
                full((H, H)), full((1, H)), full((H, 1)), full((1, 1)),
            ],
            out_specs=[eblk(H), sblk],
            out_shape=[
                jax.ShapeDtypeStruct((E2, H), jnp.float32),
                jax.ShapeDtypeStruct((ge, CR, CH), jnp.float32),
            ],
        )(z1, edge_attr[sl], geo, wc,
          Wd, be1p, We2.astype(bf),
          be2.reshape(1, H), Wx1.astype(bf), bx1.reshape(1, H),
          Wx2.astype(bf), bx2.reshape(1, 1))
        cw = cwm.reshape(nchunk2, CH)

        msgp, cdparts = scatterk(mij, cw, geo, dsth, zh, zx)
        msgps.append(msgp)
        cdps.append(cdparts)

    CW3 = NP3 // 8
    cdall = jnp.concatenate(cdps, axis=0)
    cdsum = pl.pallas_call(
        _cred_body,
        grid=(8,),
        in_specs=[pl.BlockSpec((NSPLIT * NW, CW3), lambda i: (0, i))],
        out_specs=pl.BlockSpec((1, 1, CW3), lambda i: (i, 0, 0)),
        out_shape=jax.ShapeDtypeStruct((8, 1, CW3), jnp.float32),
    )(cdall)
    cda = cdsum.reshape(NP3)[:N * 3].reshape(N, 3)

    fm2 = fixed_mask.astype(jnp.int32).reshape(N, 1)
    mblk = pl.BlockSpec((NC, NT, H), lambda i: (0, i, 0))
    h_new, xn = pl.pallas_call(
        _node_body,
        grid=(gn,),
        in_specs=[
            nblk(H), mblk, mblk,
            nblk(3), nblk(3), nblk(1),
            full((H, H)), full((H, H)), full((1, H)), full((H, H)),
            full((1, H)),
        ],
        out_specs=[nblk(H), nblk(3)],
        out_shape=[
            jax.ShapeDtypeStruct((N, H), jnp.float32),
            jax.ShapeDtypeStruct((N, 3), jnp.float32),
        ],
    )(h, msgps[0], msgps[1], cda, x, fm2,
      Wh1[:H].astype(bf), Wh1[H:].astype(bf), bh1.reshape(1, H),
      Wh2.astype(bf), bh2.reshape(1, H))

    return h_new, xn


# speedup vs baseline: 1.0753x; 1.0753x over previous
import functools

import numpy as np
import jax
import jax.numpy as jnp
from jax import lax
from jax.experimental import pallas as pl
from jax.experimental.pallas import tpu as pltpu
from jax.experimental.pallas import tpu_sc as plsc

NC = 2
NS = 16
NW = NC * NS
CH = 128
L = 16


def _pack_perm(H):
    perm = np.empty(H, np.int64)
    for g in range(H // 32):
        for i in range(16):
            perm[32 * g + i] = 32 * g + 2 * i
            perm[32 * g + 16 + i] = 32 * g + 2 * i + 1
    inv = np.argsort(perm)
    return perm, inv


def _gather_kernel(E, N, H, off):
    nchunk = E // CH
    iters = -(-nchunk // NW)
    mesh = plsc.VectorSubcoreMesh(core_axis_name="c", subcore_axis_name="s")

    @functools.partial(
        pl.kernel,
        mesh=mesh,
        out_type=(
            jax.ShapeDtypeStruct((E, H), jnp.float32),
            jax.ShapeDtypeStruct((nchunk, 4, CH), jnp.float32),
        ),
        scratch_types=[
            pltpu.VMEM((N * 3,), jnp.float32),
            pltpu.VMEM((CH,), jnp.int32),
            pltpu.VMEM((CH,), jnp.int32),
            pltpu.VMEM((CH,), jnp.int32),
            pltpu.VMEM((CH,), jnp.int32),
            pltpu.VMEM((CH, H), jnp.float32),
            pltpu.VMEM((CH, H), jnp.float32),
            pltpu.VMEM((CH, H), jnp.float32),
            pltpu.VMEM((CH, H), jnp.float32),
            pltpu.VMEM((4, CH), jnp.float32),
            pltpu.SemaphoreType.DMA,
            pltpu.SemaphoreType.DMA,
            pltpu.SemaphoreType.DMA,
            pltpu.SemaphoreType.DMA,
        ],
        compiler_params=pltpu.CompilerParams(needs_layout_passes=False),
    )
    def k(p_hbm, q_hbm, x_hbm, src_hbm, dst_hbm,
          z1_out, geo_out,
          x_v, src_v0, dst_v0, src_v1, dst_v1,
          z1_v0, qd_v0, z1_v1, qd_v1,
          geo_v, s0, s1, s2, s3):
        wid = lax.axis_index("s") * NC + lax.axis_index("c")
        pltpu.sync_copy(x_hbm, x_v)
        bufs = ((src_v0, dst_v0, z1_v0, qd_v0, s0, s1),
                (src_v1, dst_v1, z1_v1, qd_v1, s2, s3))

        def issue(kk, sv, dv, zv, qv, sp, sq):
            base = kk * CH
            pltpu.sync_copy(src_hbm.at[pl.ds(off + base, CH)], sv)
            pltpu.sync_copy(dst_hbm.at[pl.ds(off + base, CH)], dv)
            pltpu.async_copy(p_hbm.at[sv], zv, sp)
            pltpu.async_copy(q_hbm.at[dv], qv, sq)

        @pl.when(wid < nchunk)
        def _():
            issue(wid, *bufs[0])

        def grp(g, carry):
            for b in (0, 1):
                j = g * 2 + b
                kk = wid + j * NW

                @pl.when(kk < nchunk)
                def _(b=b, kk=kk):
                    sv, dv, zv, qv, sp, sq = bufs[b]
                    kk2 = kk + NW

                    @pl.when(kk2 < nchunk)
                    def _():
                        issue(kk2, *bufs[1 - b])

                    base = kk * CH
                    for i in range(CH // L):
                        sl = pl.ds(i * L, L)
                        srcs = sv[sl] * 3
                        dsts = dv[sl] * 3
                        d = []
                        for c in range(3):
                            xs_c = plsc.load_gather(x_v, [srcs + c])
                            xd_c = plsc.load_gather(x_v, [dsts + c])
                            d.append(xs_c - xd_c)
                        geo_v[0, sl] = (d[0] * d[0] + d[1] * d[1]
                                        + d[2] * d[2]) / 100.0
                        geo_v[1, sl] = d[0]
                        geo_v[2, sl] = d[1]
                        geo_v[3, sl] = d[2]
                    pltpu.make_async_copy(p_hbm.at[sv], zv, sp).wait()
                    pltpu.make_async_copy(q_hbm.at[dv], qv, sq).wait()

                    def addpack(r, carry2):
                        for gg in range(H // L):
                            sl2 = pl.ds(L * gg, L)
                            plsc.addupdate(zv.at[r, sl2], qv[r, sl2])
                        return carry2

                    lax.fori_loop(0, CH, addpack, 0)
                    pltpu.sync_copy(zv, z1_out.at[pl.ds(base, CH)])
                    pltpu.sync_copy(geo_v, geo_out.at[kk])

            return carry

        lax.fori_loop(0, (iters + 1) // 2, grp, 0)

    return k


def _scatter_kernel(E, N, H, off):
    nchunk = E // CH
    iters = -(-nchunk // NW)
    rpt = (N // NS) // 8 * 8
    tail = N - rpt * NS
    NP3 = -(-(N * 3) // 1024) * 1024
    mesh = plsc.VectorSubcoreMesh(core_axis_name="c", subcore_axis_name="s")

    @functools.partial(
        pl.kernel,
        mesh=mesh,
        out_type=(
            jax.ShapeDtypeStruct((NC, N, H), jnp.float32),
            jax.ShapeDtypeStruct((NW, NP3), jnp.float32),
        ),
        scratch_types=[
            pltpu.VMEM((CH,), jnp.int32),
            pltpu.VMEM((CH,), jnp.int32),
            pltpu.VMEM((CH, H), jnp.float32),
            pltpu.VMEM((CH,), jnp.float32),
            pltpu.VMEM((CH,), jnp.float32),
            pltpu.VMEM((4, CH), jnp.float32),
            pltpu.VMEM((4, CH), jnp.float32),
            pltpu.VMEM((NP3,), jnp.float32),
            pltpu.VMEM_SHARED((N, H), jnp.float32),
            pltpu.SemaphoreType.DMA,
            pltpu.SemaphoreType.DMA,
            pltpu.SemaphoreType.DMA,
            pltpu.SemaphoreType.DMA,
            pltpu.SemaphoreType.DMA,
            pltpu.SemaphoreType.DMA,
            pltpu.SemaphoreType.DMA,
        ],
        compiler_params=pltpu.CompilerParams(needs_layout_passes=False),
    )
    def k(m_hbm, cw_hbm, geo_hbm, dst_hbm, z_h_hbm, z_x_hbm,
          msg_out, cda_out,
          dst_v0, dst_v1, m_v, cw_v0, cw_v1, geo_v0, geo_v1,
          acc_v, msg_sh, sd0, sc0, sg0, sd1, sc1, sg1, sm):
        cid = lax.axis_index("c")
        sid = lax.axis_index("s")
        wid = sid * NC + cid
        rows = pl.ds(sid * rpt, rpt)
        trows = pl.ds(rpt * NS, tail)
        pltpu.sync_copy(z_h_hbm.at[rows], msg_sh.at[rows])

        @pl.when(sid == 0)
        def _():
            pltpu.sync_copy(z_h_hbm.at[trows], msg_sh.at[trows])

        pltpu.sync_copy(z_x_hbm, acc_v)
        plsc.subcore_barrier()
        bufs = ((dst_v0, cw_v0, geo_v0, sd0, sc0, sg0),
                (dst_v1, cw_v1, geo_v1, sd1, sc1, sg1))

        def issue(kk, dv, cv, gv, sd, sc, sg):
            base = kk * CH
            pltpu.async_copy(dst_hbm.at[pl.ds(off + base, CH)], dv, sd)
            pltpu.async_copy(cw_hbm.at[kk], cv, sc)
            pltpu.async_copy(geo_hbm.at[kk], gv, sg)

        def issue_m(kk):
            pltpu.async_copy(m_hbm.at[pl.ds(kk * CH, CH)], m_v, sm)

        @pl.when(wid < nchunk)
        def _():
            issue(wid, *bufs[0])
            issue_m(wid)

        def grp(g, carry):
            for b in (0, 1):
                j = g * 2 + b
                kk = wid + j * NW

                @pl.when(kk < nchunk)
                def _(b=b, kk=kk):
                    dv, cv, gv, sd, sc, sg = bufs[b]
                    kk2 = kk + NW

                    @pl.when(kk2 < nchunk)
                    def _():
                        issue(kk2, *bufs[1 - b])

                    base = kk * CH
                    pltpu.make_async_copy(
                        dst_hbm.at[pl.ds(off + base, CH)], dv, sd).wait()
                    pltpu.make_async_copy(cw_hbm.at[kk], cv, sc).wait()
                    pltpu.make_async_copy(geo_hbm.at[kk], gv, sg).wait()
                    pltpu.make_async_copy(
                        m_hbm.at[pl.ds(base, CH)], m_v, sm).wait()
                    pltpu.sync_copy(m_v, msg_sh.at[dv], add=True)

                    @pl.when(kk2 < nchunk)
                    def _():
                        issue_m(kk2)

                    for i in range(CH // L):
                        sl = pl.ds(i * L, L)
                        dsts = dv[sl] * 3
                        cws = cv[sl]
                        plsc.addupdate_scatter(acc_v, [dsts],
                                               gv[1, sl] * cws)
                        plsc.addupdate_scatter(acc_v, [dsts + 1],
                                               gv[2, sl] * cws)
                        plsc.addupdate_scatter(acc_v, [dsts + 2],
                                               gv[3, sl] * cws)

            return carry

        lax.fori_loop(0, (iters + 1) // 2, grp, 0)
        pltpu.sync_copy(acc_v, cda_out.at[wid])
        plsc.subcore_barrier()
        pltpu.sync_copy(msg_sh.at[rows], msg_out.at[cid].at[rows])

        @pl.when(sid == 0)
        def _():
            pltpu.sync_copy(msg_sh.at[trows], msg_out.at[cid].at[trows])

    return k


def _prep_body(h, Wa, Wb, p_out, q_out):
    p_out[...] = jnp.dot(h[...].astype(jnp.bfloat16), Wa[...],
                         preferred_element_type=jnp.float32)
    q_out[...] = jnp.dot(h[...].astype(jnp.bfloat16), Wb[...],
                         preferred_element_type=jnp.float32)


def _edge_mlp_body(z1, ea, dist, wc, Wd, be1, We2,
                   be2, Wx1, bx1, Wx2, bx2, m_out, cw_out):
    bf = jnp.bfloat16
    cr, ch = dist.shape[0], dist.shape[2]
    et = z1.shape[0]
    rows25 = lax.broadcasted_iota(jnp.int32, (et, cr), 0)
    cols25 = lax.broadcasted_iota(jnp.int32, (et, cr), 1)
    P1 = (rows25 // ch == cols25).astype(jnp.float32)
    rows1 = lax.broadcasted_iota(jnp.int32, (et, ch), 0)
    cols1 = lax.broadcasted_iota(jnp.int32, (et, ch), 1)
    Q = (rows1 % ch == cols1).astype(jnp.float32)
    rowsT = lax.broadcasted_iota(jnp.int32, (cr, et), 0)
    colsT = lax.broadcasted_iota(jnp.int32, (cr, et), 1)
    P1T = (colsT // ch == rowsT).astype(jnp.float32)

    D = dist[...][:, 0, :]
    dist_col = jnp.sum(
        jnp.dot(P1, D, preferred_element_type=jnp.float32) * Q,
        axis=1, keepdims=True)
    z = z1[...].astype(jnp.float32)
    z = z + jnp.dot(ea[...].astype(bf), Wd[...],
                    preferred_element_type=jnp.float32)
    z = z + dist_col * wc[...] + be1[...]
    m = z * jax.nn.sigmoid(z)
    z2 = jnp.dot(m.astype(bf), We2[...],
                 preferred_element_type=jnp.float32) + be2[...]
    mij = z2 * jax.nn.sigmoid(z2)
    z3 = jnp.dot(mij.astype(bf), Wx1[...],
                 preferred_element_type=jnp.float32) + bx1[...]
    t = z3 * jax.nn.sigmoid(z3)
    z4 = jnp.dot(t.astype(bf), Wx2[...],
                 preferred_element_type=jnp.float32) + bx2[...]
    cw = jnp.tanh(z4)
    m_out[...] = mij
    cw_out[...] = jnp.dot(P1T, cw * Q,
                          preferred_element_type=jnp.float32).reshape(
                              cw_out.shape)


def _cred_body(cdpa, cdpb, out):
    out[...] = (jnp.sum(cdpa[...], axis=0)
                + jnp.sum(cdpb[...], axis=0)).reshape(out.shape)


def _node_body(h, msgpa, msgpb, cda, x3, fm, Wh1a, Wh1b, bh1, Wh2, bh2,
               hn_out, xn_out):
    bf = jnp.bfloat16
    msg = msgpa[0] + msgpa[1] + msgpb[0] + msgpb[1]
    z = (jnp.dot(h[...].astype(bf), Wh1a[...],
                 preferred_element_type=jnp.float32)
         + jnp.dot(msg.astype(bf), Wh1b[...],
                   preferred_element_type=jnp.float32)
         + bh1[...])
    t = z * jax.nn.sigmoid(z)
    hn_out[...] = h[...] + jnp.dot(t.astype(bf), Wh2[...],
                                   preferred_element_type=jnp.float32) + bh2[...]
    umask = 1.0 - fm[...].astype(jnp.float32)
    xn_out[...] = x3[...] + cda[...] * umask


def kernel(h, x, edge_index, edge_attr, fixed_mask,
           We1, be1, We2, be2, Wx1, bx1, Wx2, bx2, Wh1, bh1, Wh2, bh2):
    N, H = h.shape
    E = edge_index.shape[1]
    ED = edge_attr.shape[1]
    src = edge_index[0]
    dst = edge_index[1]
    bf = jnp.bfloat16

    full = lambda shape: pl.BlockSpec(shape, lambda i: (0,) * len(shape))

    NT = 2000
    gn = N // NT
    nblk = lambda w: pl.BlockSpec((NT, w), lambda i: (i, 0))
    Wa = We1[:H].astype(bf)
    Wb = We1[H:2 * H].astype(bf)
    P, Qm = pl.pallas_call(
        _prep_body,
        grid=(gn,),
        in_specs=[nblk(H), full((H, H)), full((H, H))],
        out_specs=[nblk(H), nblk(H)],
        out_shape=[
            jax.ShapeDtypeStruct((N, H), jnp.float32),
            jax.ShapeDtypeStruct((N, H), jnp.float32),
        ],
    )(h, Wa, Wb)

    NSPLIT = 2
    E2 = E // NSPLIT
    nchunk2 = E2 // CH
    xflat = x.reshape(-1)
    zh = jnp.zeros((N, H), jnp.float32)
    NP3 = -(-(N * 3) // 1024) * 1024
    zx = jnp.zeros((NP3,), jnp.float32)

    wc = We1[2 * H:2 * H + 1]
    Wd = We1[2 * H + 1:].astype(bf)
    be1p = be1.reshape(1, H)
    ET = 3200
    ge = E2 // ET
    CR = ET // CH
    eblk = lambda w: pl.BlockSpec((ET, w), lambda i: (i, 0))
    sblk = pl.BlockSpec((1, CR, CH), lambda i: (i, 0, 0))

    msgps, cdps = [], []
    for half in range(NSPLIT):
        off = half * E2

        z1, geo = _gather_kernel(E2, N, H, off)(P, Qm, xflat, src, dst)

        gblk = pl.BlockSpec((CR, 4, CH), lambda i: (i, 0, 0))
        goff = half * ge
        mij, cwm = pl.pallas_call(
            _edge_mlp_body,
            grid=(ge,),
            in_specs=[
                eblk(H),
                pl.BlockSpec((ET, ED), lambda i, goff=goff: (i + goff, 0)),
                gblk,
                full((1, H)), full((ED, H)),
                full((1, H)), full((H, H)), full((1, H)),
                full((H, H)), full((1, H)), full((H, 1)), full((1, 1)),
            ],
            out_specs=[eblk(H), sblk],
            out_shape=[
                jax.ShapeDtypeStruct((E2, H), jnp.float32),
                jax.ShapeDtypeStruct((ge, CR, CH), jnp.float32),
            ],
        )(z1, edge_attr, geo, wc,
          Wd, be1p, We2.astype(bf),
          be2.reshape(1, H), Wx1.astype(bf), bx1.reshape(1, H),
          Wx2.astype(bf), bx2.reshape(1, 1))
        cw = cwm.reshape(nchunk2, CH)

        msgp, cdparts = _scatter_kernel(E2, N, H, off)(
            mij, cw, geo, dst, zh, zx)
        msgps.append(msgp)
        cdps.append(cdparts)

    CW3 = NP3 // 8
    cdsum = pl.pallas_call(
        _cred_body,
        grid=(8,),
        in_specs=[pl.BlockSpec((NW, CW3), lambda i: (0, i)),
                  pl.BlockSpec((NW, CW3), lambda i: (0, i))],
        out_specs=pl.BlockSpec((1, 1, CW3), lambda i: (i, 0, 0)),
        out_shape=jax.ShapeDtypeStruct((8, 1, CW3), jnp.float32),
    )(cdps[0], cdps[1])
    cda = cdsum.reshape(NP3)[:N * 3].reshape(N, 3)

    fm2 = fixed_mask.astype(jnp.int32).reshape(N, 1)
    mblk = pl.BlockSpec((NC, NT, H), lambda i: (0, i, 0))
    h_new, xn = pl.pallas_call(
        _node_body,
        grid=(gn,),
        in_specs=[
            nblk(H), mblk, mblk,
            nblk(3), nblk(3), nblk(1),
            full((H, H)), full((H, H)), full((1, H)), full((H, H)),
            full((1, H)),
        ],
        out_specs=[nblk(H), nblk(3)],
        out_shape=[
            jax.ShapeDtypeStruct((N, H), jnp.float32),
            jax.ShapeDtypeStruct((N, 3), jnp.float32),
        ],
    )(h, msgps[0], msgps[1], cda, x, fm2,
      Wh1[:H].astype(bf), Wh1[H:].astype(bf), bh1.reshape(1, H),
      Wh2.astype(bf), bh2.reshape(1, H))

    return h_new, xn
